# Initial kernel scaffold; baseline (speedup 1.0000x reference)
import jax
import jax.numpy as jnp
from jax.experimental import pallas as pl


def kernel(fmri, token, token_embedding, ctx_text, ctx_img, text_prompts, img_prompts):
    raise NotImplementedError("write your pallas kernel here")


# speedup vs baseline: 1.6531x; 1.6531x over previous
import functools

import jax
import jax.numpy as jnp
from jax import lax
from jax.experimental import pallas as pl
from jax.experimental.pallas import tpu as pltpu
from jax.experimental.pallas import tpu_sc as plsc

PROMPT_LENGTH = 2
CTX_LEN = 77
BATCH = 4096
VOCAB = 49408
D_TEXT = 512

NUM_CORES = 2
NUM_SUBCORES = 16
NUM_WORKERS = NUM_CORES * NUM_SUBCORES

TOTAL_ROWS = BATCH * CTX_LEN
ROWS_PER_WORKER = TOTAL_ROWS // NUM_WORKERS
CHUNK = 88
CHUNKS_PER_WORKER = ROWS_PER_WORKER // CHUNK
IDX_ROWS = TOTAL_ROWS // CHUNK


def _build_gather():
    mesh = plsc.VectorSubcoreMesh(
        core_axis_name="c",
        subcore_axis_name="s",
        num_cores=NUM_CORES,
        num_subcores=NUM_SUBCORES,
    )

    @functools.partial(
        pl.kernel,
        mesh=mesh,
        out_type=jax.ShapeDtypeStruct((TOTAL_ROWS, D_TEXT), jnp.float32),
        scratch_types=[
            pltpu.VMEM((CHUNKS_PER_WORKER, CHUNK), jnp.int32),
            pltpu.VMEM((CHUNK, D_TEXT), jnp.float32),
            pltpu.SemaphoreType.DMA,
        ],
    )
    def gather_kernel(table_hbm, idx_hbm, out_hbm, idx_v, rows_v, sem):
        wid = lax.axis_index("s") * NUM_CORES + lax.axis_index("c")
        r0 = wid * ROWS_PER_WORKER
        pltpu.sync_copy(idx_hbm.at[pl.ds(wid * CHUNKS_PER_WORKER, CHUNKS_PER_WORKER)],
                        idx_v)

        def body(c, carry):
            pltpu.async_copy(table_hbm.at[idx_v.at[c]], rows_v, sem).wait()
            pltpu.sync_copy(rows_v, out_hbm.at[pl.ds(r0 + c * CHUNK, CHUNK)])
            return carry

        lax.fori_loop(0, CHUNKS_PER_WORKER, body, 0)

    return gather_kernel


_gather = _build_gather()

_PROMPT_IDS = jnp.arange(VOCAB, VOCAB + PROMPT_LENGTH, dtype=jnp.int32)


@jax.jit
def kernel(fmri, token, token_embedding, ctx_text, ctx_img, text_prompts, img_prompts):
    table = jnp.concatenate([token_embedding, ctx_text], axis=0)
    idx = token[:, 0, :].astype(jnp.int32)
    idx = idx.at[:, 1:1 + PROMPT_LENGTH].set(_PROMPT_IDS[None, :])
    idx = idx.reshape(IDX_ROWS, CHUNK)
    texts = _gather(table, idx)
    texts = texts.reshape(BATCH, CTX_LEN, D_TEXT)
    return (fmri, texts, ctx_img, text_prompts, img_prompts)


# speedup vs baseline: 1.6610x; 1.0047x over previous
import functools

import jax
import jax.numpy as jnp
from jax import lax
from jax.experimental import pallas as pl
from jax.experimental.pallas import tpu as pltpu
from jax.experimental.pallas import tpu_sc as plsc

PROMPT_LENGTH = 2
CTX_LEN = 77
BATCH = 4096
VOCAB = 49408
D_TEXT = 512

NUM_CORES = 2
NUM_SUBCORES = 16
NUM_WORKERS = NUM_CORES * NUM_SUBCORES

TOTAL_ROWS = BATCH * CTX_LEN
ROWS_PER_WORKER = TOTAL_ROWS // NUM_WORKERS
CHUNK = 112
CHUNKS_PER_WORKER = ROWS_PER_WORKER // CHUNK
IDX_ROWS = TOTAL_ROWS // CHUNK
HALF_STEPS = CHUNKS_PER_WORKER // 2


def _build_gather():
    mesh = plsc.VectorSubcoreMesh(
        core_axis_name="c",
        subcore_axis_name="s",
        num_cores=NUM_CORES,
        num_subcores=NUM_SUBCORES,
    )

    @functools.partial(
        pl.kernel,
        mesh=mesh,
        out_type=jax.ShapeDtypeStruct((TOTAL_ROWS, D_TEXT), jnp.float32),
        scratch_types=[
            pltpu.VMEM((CHUNKS_PER_WORKER, CHUNK), jnp.int32),
            pltpu.VMEM((CHUNK, D_TEXT), jnp.float32),
            pltpu.VMEM((CHUNK, D_TEXT), jnp.float32),
            pltpu.SemaphoreType.DMA,
            pltpu.SemaphoreType.DMA,
            pltpu.SemaphoreType.DMA,
            pltpu.SemaphoreType.DMA,
        ],
    )
    def gather_kernel(table_hbm, idx_hbm, out_hbm,
                      idx_v, rows0, rows1, gsem0, gsem1, ssem0, ssem1):
        wid = lax.axis_index("s") * NUM_CORES + lax.axis_index("c")
        r0 = wid * ROWS_PER_WORKER
        pltpu.sync_copy(idx_hbm.at[pl.ds(wid * CHUNKS_PER_WORKER, CHUNKS_PER_WORKER)],
                        idx_v)

        def g_issue(cc, rv, gs):
            pltpu.async_copy(table_hbm.at[idx_v.at[cc]], rv, gs)

        def g_wait(rv, gs):
            pltpu.make_async_copy(table_hbm.at[pl.ds(0, CHUNK)], rv, gs).wait()

        def s_issue(cc, rv, ss):
            pltpu.async_copy(rv, out_hbm.at[pl.ds(r0 + cc * CHUNK, CHUNK)], ss)

        def s_wait(rv, ss):
            pltpu.make_async_copy(rv, out_hbm.at[pl.ds(r0, CHUNK)], ss).wait()

        g_issue(0, rows0, gsem0)
        g_wait(rows0, gsem0)
        s_issue(0, rows0, ssem0)
        g_issue(1, rows1, gsem1)
        g_wait(rows1, gsem1)
        s_issue(1, rows1, ssem1)
        s_wait(rows0, ssem0)
        g_issue(2, rows0, gsem0)

        def body(c2, carry):
            cc0 = 2 * c2
            cc1 = cc0 + 1
            g_wait(rows0, gsem0)
            s_issue(cc0, rows0, ssem0)
            s_wait(rows1, ssem1)
            g_issue(cc1, rows1, gsem1)
            g_wait(rows1, gsem1)
            s_issue(cc1, rows1, ssem1)
            s_wait(rows0, ssem0)
            g_issue(cc0 + 2, rows0, gsem0)
            return carry

        lax.fori_loop(1, HALF_STEPS - 1, body, 0)

        cc0 = 2 * (HALF_STEPS - 1)
        g_wait(rows0, gsem0)
        s_issue(cc0, rows0, ssem0)
        s_wait(rows1, ssem1)
        g_issue(cc0 + 1, rows1, gsem1)
        g_wait(rows1, gsem1)
        s_issue(cc0 + 1, rows1, ssem1)
        s_wait(rows0, ssem0)
        s_wait(rows1, ssem1)

    return gather_kernel


_gather = _build_gather()

_PROMPT_IDS = jnp.arange(VOCAB, VOCAB + PROMPT_LENGTH, dtype=jnp.int32)


@jax.jit
def kernel(fmri, token, token_embedding, ctx_text, ctx_img, text_prompts, img_prompts):
    table = jnp.concatenate([token_embedding, ctx_text], axis=0)
    idx = token[:, 0, :].astype(jnp.int32)
    idx = idx.at[:, 1:1 + PROMPT_LENGTH].set(_PROMPT_IDS[None, :])
    idx = idx.reshape(IDX_ROWS, CHUNK)
    texts = _gather(table, idx)
    texts = texts.reshape(BATCH, CTX_LEN, D_TEXT)
    return (fmri, texts, ctx_img, text_prompts, img_prompts)


# speedup vs baseline: 2.0006x; 1.2045x over previous
import functools

import jax
import jax.numpy as jnp
from jax import lax
from jax.experimental import pallas as pl
from jax.experimental.pallas import tpu as pltpu
from jax.experimental.pallas import tpu_sc as plsc

PROMPT_LENGTH = 2
CTX_LEN = 77
BATCH = 4096
VOCAB = 49408
D_TEXT = 512

NUM_CORES = 2
NUM_SUBCORES = 16
NUM_WORKERS = NUM_CORES * NUM_SUBCORES

TOTAL_ROWS = BATCH * CTX_LEN
ROWS_PER_WORKER = TOTAL_ROWS // NUM_WORKERS
CHUNK = 112
CHUNKS_PER_WORKER = ROWS_PER_WORKER // CHUNK
IDX_ROWS = TOTAL_ROWS // CHUNK
HALF_STEPS = CHUNKS_PER_WORKER // 2
LANES = 16
VECS_PER_ROW = D_TEXT // LANES


def _build_gather():
    mesh = plsc.VectorSubcoreMesh(
        core_axis_name="c",
        subcore_axis_name="s",
        num_cores=NUM_CORES,
        num_subcores=NUM_SUBCORES,
    )

    @functools.partial(
        pl.kernel,
        mesh=mesh,
        out_type=jax.ShapeDtypeStruct((IDX_ROWS, CHUNK, D_TEXT), jnp.float32),
        scratch_types=[
            pltpu.VMEM((CHUNKS_PER_WORKER, CHUNK), jnp.int32),
            pltpu.VMEM((CHUNK, D_TEXT), jnp.float32),
            pltpu.VMEM((CHUNK, D_TEXT), jnp.float32),
            pltpu.VMEM((PROMPT_LENGTH, D_TEXT), jnp.float32),
            pltpu.SemaphoreType.DMA,
            pltpu.SemaphoreType.DMA,
            pltpu.SemaphoreType.DMA,
            pltpu.SemaphoreType.DMA,
        ],
    )
    def gather_kernel(table_hbm, idx_hbm, ctx_hbm, out_hbm,
                      idx_v, rows0, rows1, ctx_v,
                      gsem0, gsem1, ssem0, ssem1):
        wid = lax.axis_index("s") * NUM_CORES + lax.axis_index("c")
        c0 = wid * CHUNKS_PER_WORKER
        pltpu.sync_copy(idx_hbm.at[pl.ds(c0, CHUNKS_PER_WORKER)], idx_v)
        pltpu.sync_copy(ctx_hbm, ctx_v)

        def g_issue(cc, rv, gs):
            pltpu.async_copy(table_hbm.at[idx_v.at[cc]], rv, gs)

        def g_wait(rv, gs):
            pltpu.make_async_copy(table_hbm.at[pl.ds(0, CHUNK)], rv, gs).wait()

        def s_issue(cc, rv, ss):
            pltpu.async_copy(rv, out_hbm.at[c0 + cc], ss)

        def s_wait(rv, ss):
            pltpu.make_async_copy(rv, out_hbm.at[c0], ss).wait()

        def write_pair(rv, o):
            for p in range(PROMPT_LENGTH):
                for k in range(VECS_PER_ROW):
                    rv[o + p, pl.ds(k * LANES, LANES)] = (
                        ctx_v[p, pl.ds(k * LANES, LANES)])

        def fix_prompts(cc, rv):
            s = (c0 + cc) * CHUNK
            m = lax.rem(s, CTX_LEN)
            o1 = lax.rem(CTX_LEN + 1 - m, CTX_LEN)
            write_pair(rv, o1)

            @pl.when(o1 + CTX_LEN + PROMPT_LENGTH <= CHUNK)
            def _():
                write_pair(rv, o1 + CTX_LEN)

        def finish(cc, rv, gs, ss):
            g_wait(rv, gs)
            fix_prompts(cc, rv)
            s_issue(cc, rv, ss)

        g_issue(0, rows0, gsem0)
        finish(0, rows0, gsem0, ssem0)
        g_issue(1, rows1, gsem1)
        finish(1, rows1, gsem1, ssem1)
        s_wait(rows0, ssem0)
        g_issue(2, rows0, gsem0)

        def body(c2, carry):
            cc0 = 2 * c2
            cc1 = cc0 + 1
            finish(cc0, rows0, gsem0, ssem0)
            s_wait(rows1, ssem1)
            g_issue(cc1, rows1, gsem1)
            finish(cc1, rows1, gsem1, ssem1)
            s_wait(rows0, ssem0)
            g_issue(cc0 + 2, rows0, gsem0)
            return carry

        lax.fori_loop(1, HALF_STEPS - 1, body, 0)

        cc0 = 2 * (HALF_STEPS - 1)
        finish(cc0, rows0, gsem0, ssem0)
        s_wait(rows1, ssem1)
        g_issue(cc0 + 1, rows1, gsem1)
        finish(cc0 + 1, rows1, gsem1, ssem1)
        s_wait(rows0, ssem0)
        s_wait(rows1, ssem1)

    return gather_kernel


_gather = _build_gather()


@jax.jit
def kernel(fmri, token, token_embedding, ctx_text, ctx_img, text_prompts, img_prompts):
    idx = token[:, 0, :].astype(jnp.int32).reshape(IDX_ROWS, CHUNK)
    texts = _gather(token_embedding, idx, ctx_text)
    texts = texts.reshape(BATCH, CTX_LEN, D_TEXT)
    return (fmri, texts, ctx_img, text_prompts, img_prompts)
